# Initial kernel scaffold; baseline (speedup 1.0000x reference)
import functools

import jax
import jax.numpy as jnp
from jax import lax
from jax.experimental import pallas as pl
from jax.experimental.pallas import tpu as pltpu
from jax.experimental.pallas import tpu_sc as plsc

POOL = 7
CELLS = POOL * POOL
SCALE = 0.03125
B, C, H, W = 2, 256, 25, 25
N = 1000
NC, NS = 2, 16
NW = NC * NS
CCHUNKS = 4
CCH = C // CCHUNKS
NVREG = CCH // 16
GROUPS = NW // CCHUNKS
RPG = N // GROUPS
STAGE = CCH * CELLS


def _div7(n):
    return (n * 9363) >> 16


def _rnd_scaled(v):
    x = v * SCALE
    t = x.astype(jnp.int32)
    f = x - t.astype(jnp.float32)
    up = (f > 0.5).astype(jnp.int32)
    half_odd = ((f == 0.5) & ((t & 1) == 1)).astype(jnp.int32)
    return t + up + half_odd


def _roipool_sc(out_words):
    mesh = plsc.VectorSubcoreMesh(core_axis_name="c", subcore_axis_name="s")

    @functools.partial(
        pl.kernel,
        out_type=jax.ShapeDtypeStruct((out_words,), jnp.float32),
        mesh=mesh,
        scratch_types=[
            pltpu.VMEM((B * H * W * CCH,), jnp.float32),
            pltpu.VMEM((RPG * 8,), jnp.float32),
            pltpu.VMEM((2 * STAGE,), jnp.float32),
            pltpu.SemaphoreType.DMA,
        ],
    )
    def k(feat_hbm, roi_hbm, out_hbm, feat_l, roi_l, stage, sem):
        wid = lax.axis_index("s") * NC + lax.axis_index("c")
        chunk = wid & 3
        grp = wid >> 2

        pltpu.sync_copy(feat_hbm.at[chunk], feat_l)
        pltpu.sync_copy(roi_hbm.at[pl.ds(grp * (RPG * 8), RPG * 8)], roi_l)

        lane49 = lax.iota(jnp.int32, 16) * CELLS
        neginf = jnp.full((16,), -jnp.inf, jnp.float32)

        def roi_body(i, _):
            base = i * 8
            x1 = _rnd_scaled(roi_l[base])
            y1 = _rnd_scaled(roi_l[base + 1])
            x2 = _rnd_scaled(roi_l[base + 2])
            y2 = _rnd_scaled(roi_l[base + 3])
            bat = roi_l[base + 4].astype(jnp.int32)
            rw = jnp.maximum(x2 - x1 + 1, 1)
            rh = jnp.maximum(y2 - y1 + 1, 1)
            pixbase = bat * (H * W)
            buf = (i & 1) * STAGE

            @pl.when(i >= 2)
            def _():
                pltpu.make_async_copy(
                    stage.at[pl.ds(buf, STAGE)],
                    out_hbm.at[pl.ds(0, STAGE)],
                    sem,
                ).wait()

            for ph in range(POOL):
                hs = jnp.clip(_div7(ph * rh) + y1, 0, H)
                he = jnp.clip(_div7((ph + 1) * rh + 6) + y1, 0, H)
                for pw in range(POOL):
                    ws = jnp.clip(_div7(pw * rw) + x1, 0, W)
                    we = jnp.clip(_div7((pw + 1) * rw + 6) + x1, 0, W)

                    def h_body(h, acc):
                        rowoff = (pixbase + h * W) * CCH

                        def w_body(w, acc):
                            off = rowoff + w * CCH
                            return tuple(
                                jnp.maximum(
                                    acc[v], feat_l[pl.ds(off + v * 16, 16)]
                                )
                                for v in range(NVREG)
                            )

                        return lax.fori_loop(ws, we, w_body, acc)

                    acc = lax.fori_loop(hs, he, h_body, (neginf,) * NVREG)
                    empty = (he <= hs) | (we <= ws)
                    cell = ph * POOL + pw
                    for v in range(NVREG):
                        val = jnp.where(empty, 0.0, acc[v])
                        plsc.store_scatter(
                            stage,
                            [lane49 + (buf + v * 16 * CELLS + cell)],
                            val,
                        )

            out_off = ((grp * RPG + i) * CCHUNKS + chunk) * STAGE
            pltpu.async_copy(
                stage.at[pl.ds(buf, STAGE)],
                out_hbm.at[pl.ds(out_off, STAGE)],
                sem,
            )
            return 0

        lax.fori_loop(0, RPG, roi_body, 0)
        for _ in range(2):
            pltpu.make_async_copy(
                stage.at[pl.ds(0, STAGE)],
                out_hbm.at[pl.ds(0, STAGE)],
                sem,
            ).wait()

    return k


def kernel(feat, rois, roibatches):
    feat_r = (
        feat.transpose(0, 2, 3, 1)
        .reshape(B, H, W, CCHUNKS, CCH)
        .transpose(3, 0, 1, 2, 4)
        .reshape(CCHUNKS, B * H * W * CCH)
    )
    roi_pack = jnp.concatenate(
        [rois, roibatches.astype(jnp.float32)[:, None],
         jnp.zeros((N, 3), jnp.float32)],
        axis=1,
    ).reshape(N * 8)
    out_words = N * C * CELLS
    out = _roipool_sc(out_words)(feat_r, roi_pack)
    return out.reshape(N, C, POOL, POOL)


# speedup vs baseline: 15.6890x; 15.6890x over previous
import functools

import jax
import jax.numpy as jnp
from jax import lax
from jax.experimental import pallas as pl
from jax.experimental.pallas import tpu as pltpu
from jax.experimental.pallas import tpu_sc as plsc

POOL = 7
CELLS = POOL * POOL
KWIN = 6
SCALE = 0.03125
B, C, H, W = 2, 256, 25, 25
N = 1000
NC, NS = 2, 16
NW = NC * NS
CCHUNKS = 4
CCH = C // CCHUNKS
NVREG = CCH // 16
GROUPS = NW // CCHUNKS
RPG = N // GROUPS
STAGE = CCH * CELLS


def _roipool_sc(out_words):
    mesh = plsc.VectorSubcoreMesh(core_axis_name="c", subcore_axis_name="s")

    @functools.partial(
        pl.kernel,
        out_type=jax.ShapeDtypeStruct((out_words,), jnp.float32),
        mesh=mesh,
        compiler_params=pltpu.CompilerParams(needs_layout_passes=False),
        scratch_types=[
            pltpu.VMEM((B * H * W * CCH,), jnp.float32),
            pltpu.VMEM((RPG * 32,), jnp.int32),
            pltpu.VMEM((2 * STAGE,), jnp.float32),
            pltpu.SemaphoreType.DMA,
        ],
    )
    def k(feat_hbm, roi_hbm, out_hbm, feat_l, roi_l, stage, sem):
        wid = lax.axis_index("s") * NC + lax.axis_index("c")
        chunk = wid & 3
        grp = wid >> 2

        pltpu.sync_copy(feat_hbm.at[chunk], feat_l)
        pltpu.sync_copy(roi_hbm.at[pl.ds(grp * (RPG * 32), RPG * 32)], roi_l)

        lane49 = lax.iota(jnp.int32, 16) * CELLS
        neginf = jnp.full((16,), -jnp.inf, jnp.float32)

        def roi_body(i, _):
            va = roi_l[pl.ds(i * 32, 16)]
            vb = roi_l[pl.ds(i * 32 + 16, 16)]
            bat = vb[14]
            pixbase = bat * (H * W)
            buf = (i & 1) * STAGE

            @pl.when(i >= 2)
            def _():
                pltpu.make_async_copy(
                    stage.at[pl.ds(buf, STAGE)],
                    out_hbm.at[pl.ds(0, STAGE)],
                    sem,
                ).wait()

            for ph in range(POOL):
                hs = va[ph]
                he = va[7 + ph]
                for pw in range(POOL):
                    ws = vb[pw]
                    we = vb[7 + pw]

                    def h_body(h, acc):
                        rowoff = (pixbase + h * W) * CCH

                        def w_body(w, acc):
                            off = rowoff + w * CCH
                            return tuple(
                                jnp.maximum(
                                    acc[v], feat_l[pl.ds(off + v * 16, 16)]
                                )
                                for v in range(NVREG)
                            )

                        return lax.fori_loop(ws, we, w_body, acc)

                    acc = lax.fori_loop(hs, he, h_body, (neginf,) * NVREG)
                    empty = (he <= hs) | (we <= ws)
                    cell = ph * POOL + pw
                    for v in range(NVREG):
                        val = jnp.where(empty, 0.0, acc[v])
                        plsc.store_scatter(
                            stage,
                            [lane49 + (buf + v * 16 * CELLS + cell)],
                            val,
                        )

            out_off = ((grp * RPG + i) * CCHUNKS + chunk) * STAGE
            pltpu.async_copy(
                stage.at[pl.ds(buf, STAGE)],
                out_hbm.at[pl.ds(out_off, STAGE)],
                sem,
            )
            return 0

        lax.fori_loop(0, RPG, roi_body, 0)
        for _ in range(2):
            pltpu.make_async_copy(
                stage.at[pl.ds(0, STAGE)],
                out_hbm.at[pl.ds(0, STAGE)],
                sem,
            ).wait()

    return k


def _bin_tables():
    import numpy as _np

    tf = _np.zeros((32, 8), _np.int32)
    tc = _np.zeros((32, 8), _np.int32)
    recip = _np.float32(1.0) / _np.float32(POOL)
    for rh in range(1, 32):
        b = _np.float32(rh) * recip
        for p in range(8):
            prod = _np.float32(p) * b
            tf[rh, p] = int(_np.floor(prod))
            tc[rh, p] = int(_np.ceil(prod))
    return jnp.asarray(tf.reshape(-1)), jnp.asarray(tc.reshape(-1))


def _bounds(rois, roibatches):
    r = jnp.round(lax.stop_gradient(rois) * SCALE).astype(jnp.int32)
    x1, y1, x2, y2 = r[:, 0], r[:, 1], r[:, 2], r[:, 3]
    roi_w = jnp.maximum(x2 - x1 + 1, 1)
    roi_h = jnp.maximum(y2 - y1 + 1, 1)
    tf, tc = _bin_tables()
    p = jnp.arange(POOL)
    hs = jnp.clip(jnp.take(tf, roi_h[:, None] * 8 + p[None, :]) + y1[:, None], 0, H)
    he = jnp.clip(jnp.take(tc, roi_h[:, None] * 8 + p[None, :] + 1) + y1[:, None], 0, H)
    ws = jnp.clip(jnp.take(tf, roi_w[:, None] * 8 + p[None, :]) + x1[:, None], 0, W)
    we = jnp.clip(jnp.take(tc, roi_w[:, None] * 8 + p[None, :] + 1) + x1[:, None], 0, W)
    he_c = jnp.minimum(he, hs + KWIN)
    we_c = jnp.minimum(we, ws + KWIN)
    z = jnp.zeros((N, 1), jnp.int32)
    return jnp.concatenate(
        [hs, he_c, z, z, ws, we_c, roibatches[:, None], z],
        axis=1,
    ).reshape(N * 32)


def kernel(feat, rois, roibatches):
    feat_r = (
        feat.transpose(0, 2, 3, 1)
        .reshape(B, H, W, CCHUNKS, CCH)
        .transpose(3, 0, 1, 2, 4)
        .reshape(CCHUNKS, B * H * W * CCH)
    )
    roi_pack = _bounds(rois, roibatches)
    out_words = N * C * CELLS
    out = _roipool_sc(out_words)(feat_r, roi_pack)
    return out.reshape(N, C, POOL, POOL)


# speedup vs baseline: 16.7536x; 1.0679x over previous
import functools

import jax
import jax.numpy as jnp
from jax import lax
from jax.experimental import pallas as pl
from jax.experimental.pallas import tpu as pltpu
from jax.experimental.pallas import tpu_sc as plsc

POOL = 7
CELLS = POOL * POOL
KWIN = 6
SCALE = 0.03125
B, C, H, W = 2, 256, 25, 25
N = 1000
NC, NS = 2, 16
NW = NC * NS
CCHUNKS = 4
CCH = C // CCHUNKS
NVREG = CCH // 16
GROUPS = NW // CCHUNKS
RPG = N // GROUPS
STAGE = CCH * CELLS


def _roipool_sc(out_words):
    mesh = plsc.VectorSubcoreMesh(core_axis_name="c", subcore_axis_name="s")
    half_words = out_words // NC

    @functools.partial(
        pl.kernel,
        out_type=(
            jax.ShapeDtypeStruct((half_words,), jnp.float32),
            jax.ShapeDtypeStruct((half_words,), jnp.float32),
        ),
        mesh=mesh,
        compiler_params=pltpu.CompilerParams(needs_layout_passes=False),
        scratch_types=[
            pltpu.VMEM((B * H * W * CCH,), jnp.float32),
            pltpu.VMEM((RPG * 32,), jnp.int32),
            pltpu.VMEM((2 * STAGE,), jnp.float32),
            pltpu.SemaphoreType.DMA,
        ],
    )
    def k(feat_hbm, roi_hbm, out0_hbm, out1_hbm, feat_l, roi_l, stage, sem):
        cidx = lax.axis_index("c")
        sidx = lax.axis_index("s")
        chunk = sidx & 3
        grp_local = sidx >> 2
        grp = cidx * (GROUPS // NC) + grp_local

        pltpu.sync_copy(feat_hbm.at[chunk], feat_l)
        pltpu.sync_copy(roi_hbm.at[pl.ds(grp * (RPG * 32), RPG * 32)], roi_l)

        lane49 = lax.iota(jnp.int32, 16) * CELLS
        neginf = jnp.full((16,), -jnp.inf, jnp.float32)

        def roi_body(i, _):
            va = roi_l[pl.ds(i * 32, 16)]
            vb = roi_l[pl.ds(i * 32 + 16, 16)]
            bat = vb[14]
            pixbase = bat * (H * W)
            buf = (i & 1) * STAGE

            @pl.when(i >= 2)
            def _():
                pltpu.make_async_copy(
                    stage.at[pl.ds(buf, STAGE)],
                    out0_hbm.at[pl.ds(0, STAGE)],
                    sem,
                ).wait()

            for ph in range(POOL):
                hs = va[ph]
                he = va[7 + ph]
                for pw in range(POOL):
                    ws = vb[pw]
                    we = vb[7 + pw]

                    def h_body(h, acc):
                        rowoff = (pixbase + h * W) * CCH

                        def w_body(w, acc):
                            off = rowoff + w * CCH
                            return tuple(
                                jnp.maximum(
                                    acc[v], feat_l[pl.ds(off + v * 16, 16)]
                                )
                                for v in range(NVREG)
                            )

                        return lax.fori_loop(ws, we, w_body, acc)

                    acc = lax.fori_loop(hs, he, h_body, (neginf,) * NVREG)
                    empty = (he <= hs) | (we <= ws)
                    cell = ph * POOL + pw
                    for v in range(NVREG):
                        val = jnp.where(empty, 0.0, acc[v])
                        plsc.store_scatter(
                            stage,
                            [lane49 + (buf + v * 16 * CELLS + cell)],
                            val,
                        )

            out_off = ((grp_local * RPG + i) * CCHUNKS + chunk) * STAGE

            @pl.when(cidx == 0)
            def _():
                pltpu.async_copy(
                    stage.at[pl.ds(buf, STAGE)],
                    out0_hbm.at[pl.ds(out_off, STAGE)],
                    sem,
                )

            @pl.when(cidx == 1)
            def _():
                pltpu.async_copy(
                    stage.at[pl.ds(buf, STAGE)],
                    out1_hbm.at[pl.ds(out_off, STAGE)],
                    sem,
                )

            return 0

        lax.fori_loop(0, RPG, roi_body, 0)
        for _ in range(2):
            pltpu.make_async_copy(
                stage.at[pl.ds(0, STAGE)],
                out0_hbm.at[pl.ds(0, STAGE)],
                sem,
            ).wait()

    return k


def _bin_tables():
    import numpy as _np

    tf = _np.zeros((32, 8), _np.int32)
    tc = _np.zeros((32, 8), _np.int32)
    recip = _np.float32(1.0) / _np.float32(POOL)
    for rh in range(1, 32):
        b = _np.float32(rh) * recip
        for p in range(8):
            prod = _np.float32(p) * b
            tf[rh, p] = int(_np.floor(prod))
            tc[rh, p] = int(_np.ceil(prod))
    return jnp.asarray(tf.reshape(-1)), jnp.asarray(tc.reshape(-1))


def _bounds(rois, roibatches):
    r = jnp.round(lax.stop_gradient(rois) * SCALE).astype(jnp.int32)
    x1, y1, x2, y2 = r[:, 0], r[:, 1], r[:, 2], r[:, 3]
    roi_w = jnp.maximum(x2 - x1 + 1, 1)
    roi_h = jnp.maximum(y2 - y1 + 1, 1)
    tf, tc = _bin_tables()
    p = jnp.arange(POOL)
    hs = jnp.clip(jnp.take(tf, roi_h[:, None] * 8 + p[None, :]) + y1[:, None], 0, H)
    he = jnp.clip(jnp.take(tc, roi_h[:, None] * 8 + p[None, :] + 1) + y1[:, None], 0, H)
    ws = jnp.clip(jnp.take(tf, roi_w[:, None] * 8 + p[None, :]) + x1[:, None], 0, W)
    we = jnp.clip(jnp.take(tc, roi_w[:, None] * 8 + p[None, :] + 1) + x1[:, None], 0, W)
    he_c = jnp.minimum(he, hs + KWIN)
    we_c = jnp.minimum(we, ws + KWIN)
    z = jnp.zeros((N, 1), jnp.int32)
    return jnp.concatenate(
        [hs, he_c, z, z, ws, we_c, roibatches[:, None], z],
        axis=1,
    ).reshape(N * 32)


def kernel(feat, rois, roibatches):
    feat_r = (
        feat.transpose(0, 2, 3, 1)
        .reshape(B, H, W, CCHUNKS, CCH)
        .transpose(3, 0, 1, 2, 4)
        .reshape(CCHUNKS, B * H * W * CCH)
    )
    roi_pack = _bounds(rois, roibatches)
    out_words = N * C * CELLS
    o0, o1 = _roipool_sc(out_words)(feat_r, roi_pack)
    return jnp.concatenate([o0, o1]).reshape(N, C, POOL, POOL)
